# Initial kernel scaffold; baseline (speedup 1.0000x reference)
import jax
import jax.numpy as jnp
from jax.experimental import pallas as pl


def kernel(x, gate_w, ln_g, ln_b, W1, b1, W2, b2):
    raise NotImplementedError("write your pallas kernel here")


# speedup vs baseline: 2.3441x; 2.3441x over previous
import functools

import jax
import jax.numpy as jnp
from jax.experimental import pallas as pl
from jax.experimental.pallas import tpu as pltpu

B, T, D = 1, 2048, 768
E, K, FF = 8, 2, 1024
BT = B * T


def _moe_body(x_ref, gate_ref, ln_g_ref, ln_b_ref, w1_ref, b1_ref,
              w2_ref, b2_ref, out_ref, aux_ref, xc_s, wmask_s, wtot_s):
    e = pl.program_id(0)

    @pl.when(e == 0)
    def _router():
        x = x_ref[...]
        mu = jnp.mean(x, axis=1, keepdims=True)
        var = jnp.mean((x - mu) ** 2, axis=1, keepdims=True)
        xc_s[...] = (x - mu) * jax.lax.rsqrt(var + 1e-6)
        logits = jax.lax.dot_general(
            x, gate_ref[...], (((1,), (1,)), ((), ())),
            preferred_element_type=jnp.float32)
        m = jnp.max(logits, axis=1, keepdims=True)
        ex = jnp.exp(logits - m)
        s = ex / jnp.sum(ex, axis=1, keepdims=True)
        eids = jax.lax.broadcasted_iota(jnp.int32, (BT, E), 1)
        v1 = jnp.max(s, axis=1, keepdims=True)
        i1 = jnp.min(jnp.where(s == v1, eids, E), axis=1, keepdims=True)
        s_m = jnp.where(eids == i1, -1.0, s)
        v2 = jnp.max(s_m, axis=1, keepdims=True)
        i2 = jnp.min(jnp.where(s_m == v2, eids, E), axis=1, keepdims=True)
        sel = (eids == i1) | (eids == i2)
        wmask = jnp.where(sel, s, 0.0)
        wmask_s[...] = wmask
        wtot_s[...] = jnp.sum(wmask, axis=1, keepdims=True)
        load = jnp.sum(sel.astype(jnp.float32), axis=0)
        importance = jnp.sum(s, axis=0)
        aux = jnp.sum(load * importance) * (E / (BT * BT))
        aux_ref[...] = jnp.broadcast_to(aux, (1, 1))

    xn = xc_s[...] * ln_g_ref[0] + ln_b_ref[0]
    h = jax.lax.dot_general(
        xn, w1_ref[0], (((1,), (1,)), ((), ())),
        preferred_element_type=jnp.float32)
    h = jnp.maximum(h + b1_ref[0], 0.0)
    y = jax.lax.dot_general(
        h, w2_ref[0], (((1,), (1,)), ((), ())),
        preferred_element_type=jnp.float32)
    y = y + b2_ref[0]
    lane = jax.lax.broadcasted_iota(jnp.int32, (BT, E), 1)
    w_col = jnp.sum(jnp.where(lane == e, wmask_s[...], 0.0),
                    axis=1, keepdims=True)

    @pl.when(e == 0)
    def _init():
        out_ref[...] = w_col * y + wtot_s[...] * x_ref[...]

    @pl.when(e != 0)
    def _acc():
        out_ref[...] = out_ref[...] + w_col * y


@jax.jit
def kernel(x, gate_w, ln_g, ln_b, W1, b1, W2, b2):
    xf = x.reshape(BT, D)
    out, aux = pl.pallas_call(
        _moe_body,
        grid=(E,),
        in_specs=[
            pl.BlockSpec((BT, D), lambda e: (0, 0)),
            pl.BlockSpec((E, D), lambda e: (0, 0)),
            pl.BlockSpec((1, 1, D), lambda e: (e, 0, 0)),
            pl.BlockSpec((1, 1, D), lambda e: (e, 0, 0)),
            pl.BlockSpec((1, FF, D), lambda e: (e, 0, 0)),
            pl.BlockSpec((1, 1, FF), lambda e: (e, 0, 0)),
            pl.BlockSpec((1, D, FF), lambda e: (e, 0, 0)),
            pl.BlockSpec((1, 1, D), lambda e: (e, 0, 0)),
        ],
        out_specs=[
            pl.BlockSpec((BT, D), lambda e: (0, 0)),
            pl.BlockSpec((1, 1), lambda e: (0, 0)),
        ],
        out_shape=[
            jax.ShapeDtypeStruct((BT, D), jnp.float32),
            jax.ShapeDtypeStruct((1, 1), jnp.float32),
        ],
        scratch_shapes=[
            pltpu.VMEM((BT, D), jnp.float32),
            pltpu.VMEM((BT, E), jnp.float32),
            pltpu.VMEM((BT, 1), jnp.float32),
        ],
        compiler_params=pltpu.CompilerParams(
            dimension_semantics=("arbitrary",),
        ),
    )(xf, gate_w, ln_g.reshape(E, 1, D), ln_b.reshape(E, 1, D), W1,
      b1.reshape(E, 1, FF), W2, b2.reshape(E, 1, D))
    return out.reshape(B, T, D), aux[0, 0]
